# Initial kernel scaffold; baseline (speedup 1.0000x reference)
import jax
import jax.numpy as jnp
from jax.experimental import pallas as pl


def kernel(inp_node_features, inp_adj_features, rgcn_w0, rgcn_w, node_st_w, node_st_b, edge_st_w, edge_st_b):
    raise NotImplementedError("write your pallas kernel here")


# speedup vs baseline: 1.1384x; 1.1384x over previous
import numpy as np
import jax
import jax.numpy as jnp
from jax import lax
from jax.experimental import pallas as pl
from jax.experimental.pallas import tpu as pltpu

_MAX = 38
_NP = 40
_EU = 12
_ND = 9
_BD = 4
_KC = _BD * _NP
_NH = 128
_NFL = 12


def _static_consts():
    src, dst = [], []
    for i in range(_MAX):
        start = 0 if i < _EU else i - _EU
        tot = i if i < _EU else _EU
        for j in range(tot):
            src.append(start + j)
            dst.append(i)
    src = np.asarray(src)
    dst = np.asarray(dst)
    e = src.shape[0]
    gdst = np.zeros((e, _NP), np.float32)
    gdst[np.arange(e), dst] = 1.0
    gsrc = np.zeros((e, _NP), np.float32)
    gsrc[np.arange(e), src] = 1.0
    lincl = np.tril(np.ones((_NP, _NP), np.float32))
    lincl[_MAX:, :] = 0.0
    gsrct = np.tile(gsrc, (1, _BD))
    sel = np.zeros((_KC, _BD), np.float32)
    for r in range(_BD):
        sel[r * _NP:(r + 1) * _NP, r] = 1.0
    mask_n = np.zeros((1, _NP, 1), np.float32)
    mask_n[0, :_MAX, 0] = 1.0
    return gdst, gsrc, lincl, gsrct, sel, mask_n


_GDST, _GSRC, _LINCL, _GSRCT, _SEL, _MASKN = _static_consts()
_E = _GDST.shape[0]
_BF = jnp.bfloat16
_F32 = jnp.float32


def _mm(a, b):
    return jnp.dot(a, b, preferred_element_type=_F32)


def _mm_t(a, b):
    return lax.dot_general(a, b, (((1,), (1,)), ((), ())),
                           preferred_element_type=_F32)


def _split(u):
    hi = u.astype(_BF)
    lo = (u - hi.astype(_F32)).astype(_BF)
    return hi, lo


def _xdot(g16, u):
    hi, lo = _split(u)
    return _mm(g16, hi) + _mm(g16, lo)


def _xdot_t(g16, u):
    hi, lo = _split(u)
    return _mm_t(g16, hi) + _mm_t(g16, lo)


def _body(x_ref, adjs_ref, w0st_ref, wst_ref, wn_ref, bn_ref,
          wg_ref, wd_ref, ws_ref, be_ref,
          gdst_ref, gsrc_ref, lincl_ref, gsrct_ref, sel_ref, maskn_ref,
          zn_ref, ze_ref, ldn_ref, lde_ref):
    bb = x_ref.shape[0]

    a16 = [adjs_ref[b].astype(_BF) for b in range(bb)]

    def rgcn_layer(hb16, wst16):
        out = []
        for b in range(bb):
            m = _mm(a16[b], hb16[b])
            mcat = jnp.concatenate(
                [m[r * _NP:(r + 1) * _NP, :] for r in range(_BD)], axis=1)
            out.append(jnp.maximum(_mm(mcat.astype(_BF), wst16), 0.0))
        return out

    hb = rgcn_layer([x_ref[b].astype(_BF) for b in range(bb)],
                    w0st_ref[...].astype(_BF))
    for l in range(2):
        hb = rgcn_layer([h.astype(_BF) for h in hb],
                        wst_ref[l].astype(_BF))

    h16 = [h.astype(_BF) for h in hb]
    l16 = lincl_ref[...].astype(_BF)
    cs = jnp.concatenate([_mm(l16, h16[b]) for b in range(bb)], axis=0)
    hs16 = jnp.concatenate(h16, axis=0)
    gs = cs - hs16.astype(_F32)

    stn = _mm(gs.astype(_BF), wn_ref[...].astype(_BF)) + bn_ref[0]
    s_all = jax.nn.sigmoid(stn[:, :_NFL * _ND] + 2.0)
    logs = jnp.log(s_all).reshape(bb, _NP, _NFL * _ND) * maskn_ref[...]
    ldn_ref[...] = jnp.sum(logs, axis=(1, 2)).reshape(bb, 1)
    z = x_ref[...].reshape(bb * _NP, _ND)
    for l in range(_NFL):
        z = (z * s_all[:, l * _ND:(l + 1) * _ND]
             + stn[:, _NFL * _ND + l * _ND:_NFL * _ND + (l + 1) * _ND])
    zn_ref[...] = z.reshape(bb, _NP, _ND)[:, :_MAX, :]

    u = (_mm(cs.astype(_BF), wg_ref[...].astype(_BF))
         + _mm(hs16, wd_ref[...].astype(_BF)))
    v = _mm(hs16, ws_ref[...].astype(_BF))
    gdst16 = gdst_ref[...].astype(_BF)
    gsrc16 = gsrc_ref[...].astype(_BF)
    gsrct = gsrct_ref[...]
    sel16 = sel_ref[...].astype(_BF)
    be = be_ref[0]
    lde_l = []
    for b in range(bb):
        ub = u[b * _NP:(b + 1) * _NP, :]
        vb = v[b * _NP:(b + 1) * _NP, :]
        ste = _xdot(gdst16, ub) + _xdot(gsrc16, vb) + be
        rmat = _xdot_t(gdst16, adjs_ref[b])
        rg = rmat * gsrct
        hi, lo = _split(rg)
        ze = _mm(hi, sel16) + _mm(lo, sel16)
        se_all = jax.nn.sigmoid(ste[:, :_NFL * _BD] + 2.0)
        lde_l.append(jnp.sum(jnp.log(se_all)).reshape(1, 1))
        for l in range(_NFL):
            ze = (ze * se_all[:, l * _BD:(l + 1) * _BD]
                  + ste[:, _NFL * _BD + l * _BD:_NFL * _BD + (l + 1) * _BD])
        ze_ref[b] = ze
    lde_ref[...] = jnp.concatenate(lde_l, axis=0)


def kernel(inp_node_features, inp_adj_features, rgcn_w0, rgcn_w,
           node_st_w, node_st_b, edge_st_w, edge_st_b):
    x = inp_node_features
    adj = inp_adj_features
    b = x.shape[0]
    xp = jnp.pad(x, ((0, 0), (0, _NP - _MAX), (0, 0)))
    adjs = jnp.pad(adj, ((0, 0), (0, 0), (0, _NP - _MAX), (0, _NP - _MAX)))
    adjs = adjs.reshape(b, _KC, _NP)

    w0st = rgcn_w0.reshape(_BD * _ND, _NH)
    wst = rgcn_w.reshape(2, _BD * _NH, _NH)

    wn = jnp.concatenate(
        [node_st_w[:, :, :_ND].transpose(1, 0, 2).reshape(_NH, _NFL * _ND),
         node_st_w[:, :, _ND:].transpose(1, 0, 2).reshape(_NH, _NFL * _ND)],
        axis=1)
    bn = jnp.concatenate(
        [node_st_b[:, :_ND].reshape(-1), node_st_b[:, _ND:].reshape(-1)],
        axis=0).reshape(1, 2 * _NFL * _ND)

    def epack(wslice):
        return jnp.concatenate(
            [wslice[:, :, :_BD].transpose(1, 0, 2).reshape(_NH, _NFL * _BD),
             wslice[:, :, _BD:].transpose(1, 0, 2).reshape(_NH, _NFL * _BD)],
            axis=1)

    wg = epack(edge_st_w[:, :_NH, :])
    ws = epack(edge_st_w[:, _NH:2 * _NH, :])
    wd = epack(edge_st_w[:, 2 * _NH:, :])
    be = jnp.concatenate(
        [edge_st_b[:, :_BD].reshape(-1), edge_st_b[:, _BD:].reshape(-1)],
        axis=0).reshape(1, 2 * _NFL * _BD)

    bb = 16
    grid = (b // bb,)

    def bs_b(shape):
        n = len(shape)
        return pl.BlockSpec((bb,) + shape, lambda i, n=n: (i,) + (0,) * n)

    def bs_w(shape):
        n = len(shape)
        return pl.BlockSpec(shape, lambda i, n=n: (0,) * n)

    out_shape = [
        jax.ShapeDtypeStruct((b, _MAX, _ND), jnp.float32),
        jax.ShapeDtypeStruct((b, _E, _BD), jnp.float32),
        jax.ShapeDtypeStruct((b, 1), jnp.float32),
        jax.ShapeDtypeStruct((b, 1), jnp.float32),
    ]
    out_specs = [
        bs_b((_MAX, _ND)),
        bs_b((_E, _BD)),
        pl.BlockSpec((bb, 1), lambda i: (i, 0)),
        pl.BlockSpec((bb, 1), lambda i: (i, 0)),
    ]
    in_specs = [
        bs_b((_NP, _ND)),
        bs_b((_KC, _NP)),
        bs_w(w0st.shape),
        bs_w(wst.shape),
        bs_w(wn.shape),
        bs_w(bn.shape),
        bs_w(wg.shape),
        bs_w(wd.shape),
        bs_w(ws.shape),
        bs_w(be.shape),
        bs_w(_GDST.shape),
        bs_w(_GSRC.shape),
        bs_w(_LINCL.shape),
        bs_w(_GSRCT.shape),
        bs_w(_SEL.shape),
        bs_w(_MASKN.shape),
    ]
    zn, ze, ldn, lde = pl.pallas_call(
        _body,
        grid=grid,
        in_specs=in_specs,
        out_specs=out_specs,
        out_shape=out_shape,
        compiler_params=pltpu.CompilerParams(
            dimension_semantics=("arbitrary",)),
    )(xp, adjs, w0st, wst, wn, bn, wg, wd, ws, be,
      jnp.asarray(_GDST), jnp.asarray(_GSRC), jnp.asarray(_LINCL),
      jnp.asarray(_GSRCT), jnp.asarray(_SEL), jnp.asarray(_MASKN))
    return (zn.reshape(b, -1), ze.reshape(b, -1),
            ldn.reshape(b), lde.reshape(b))


# speedup vs baseline: 1.4114x; 1.2398x over previous
import numpy as np
import jax
import jax.numpy as jnp
from jax import lax
from jax.experimental import pallas as pl
from jax.experimental.pallas import tpu as pltpu

_MAX = 38
_NP = 40
_EU = 12
_ND = 9
_BD = 4
_KC = _BD * _NP
_NH = 128
_NFL = 12


def _static_consts():
    src, dst = [], []
    for i in range(_MAX):
        start = 0 if i < _EU else i - _EU
        tot = i if i < _EU else _EU
        for j in range(tot):
            src.append(start + j)
            dst.append(i)
    src = np.asarray(src)
    dst = np.asarray(dst)
    e = src.shape[0]
    gdst = np.zeros((e, _NP), np.float32)
    gdst[np.arange(e), dst] = 1.0
    gsrc = np.zeros((e, _NP), np.float32)
    gsrc[np.arange(e), src] = 1.0
    lincl = np.tril(np.ones((_NP, _NP), np.float32))
    lincl[_MAX:, :] = 0.0
    gsrct = np.tile(gsrc, (1, _BD))
    sel = np.zeros((_KC, _BD), np.float32)
    for r in range(_BD):
        sel[r * _NP:(r + 1) * _NP, r] = 1.0
    mask_n = np.zeros((1, _NP, 1), np.float32)
    mask_n[0, :_MAX, 0] = 1.0
    return gdst, gsrc, lincl, gsrct, sel, mask_n


_GDST, _GSRC, _LINCL, _GSRCT, _SEL, _MASKN = _static_consts()
_E = _GDST.shape[0]
_BF = jnp.bfloat16
_F32 = jnp.float32


def _mm(a, b):
    return jnp.dot(a, b, preferred_element_type=_F32)


def _mm_t(a, b):
    return lax.dot_general(a, b, (((1,), (1,)), ((), ())),
                           preferred_element_type=_F32)


def _split(u):
    hi = u.astype(_BF)
    lo = (u - hi.astype(_F32)).astype(_BF)
    return hi, lo


def _xdot(g16, u):
    hi, lo = _split(u)
    return _mm(g16, hi) + _mm(g16, lo)


def _xdot_t(g16, u):
    hi, lo = _split(u)
    return _mm_t(g16, hi) + _mm_t(g16, lo)


def _body(x_ref, adjs_ref, w0st_ref, wst_ref, wn_ref, bn_ref,
          wg_ref, wd_ref, ws_ref, be_ref,
          gdst_ref, gsrc_ref, lincl_ref, gsrct_ref, sel_ref, maskn_ref,
          zn_ref, ze_ref, ldn_ref, lde_ref):
    bb = x_ref.shape[0]

    a16 = [adjs_ref[b].astype(_BF) for b in range(bb)]

    def rgcn_layer(hb16, wst16):
        out = []
        for b in range(bb):
            m = _mm(a16[b], hb16[b])
            mcat = jnp.concatenate(
                [m[r * _NP:(r + 1) * _NP, :] for r in range(_BD)], axis=1)
            out.append(jnp.maximum(_mm(mcat.astype(_BF), wst16), 0.0))
        return out

    hb = rgcn_layer([x_ref[b].astype(_BF) for b in range(bb)],
                    w0st_ref[...].astype(_BF))
    for l in range(2):
        hb = rgcn_layer([h.astype(_BF) for h in hb],
                        wst_ref[l].astype(_BF))

    h16 = [h.astype(_BF) for h in hb]
    l16 = lincl_ref[...].astype(_BF)
    cs = jnp.concatenate([_mm(l16, h16[b]) for b in range(bb)], axis=0)
    hs16 = jnp.concatenate(h16, axis=0)
    gs = cs - hs16.astype(_F32)

    stn = _mm(gs.astype(_BF), wn_ref[...].astype(_BF)) + bn_ref[0]
    s_all = jax.nn.sigmoid(stn[:, :_NFL * _ND])
    logs = jnp.log(s_all).reshape(bb, _NP, _NFL * _ND) * maskn_ref[...]
    ldn_ref[...] = jnp.sum(logs, axis=(1, 2)).reshape(bb, 1)
    z = x_ref[...].reshape(bb * _NP, _ND)
    for l in range(_NFL):
        z = (z * s_all[:, l * _ND:(l + 1) * _ND]
             + stn[:, _NFL * _ND + l * _ND:_NFL * _ND + (l + 1) * _ND])
    zn_ref[...] = z.reshape(bb, _NP, _ND)[:, :_MAX, :]

    u = (_mm(cs.astype(_BF), wg_ref[...].astype(_BF))
         + _mm(hs16, wd_ref[...].astype(_BF)))
    v = _mm(hs16, ws_ref[...].astype(_BF))
    gdst16 = gdst_ref[...].astype(_BF)
    gsrc16 = gsrc_ref[...].astype(_BF)
    gsrct = gsrct_ref[...]
    sel16 = sel_ref[...].astype(_BF)
    be = be_ref[0]
    lde_l = []
    for b in range(bb):
        ub = u[b * _NP:(b + 1) * _NP, :]
        vb = v[b * _NP:(b + 1) * _NP, :]
        ste = (_mm(gdst16, ub.astype(_BF))
               + _mm(gsrc16, vb.astype(_BF)) + be)
        rmat = _xdot_t(gdst16, adjs_ref[b])
        rg = rmat * gsrct
        hi, lo = _split(rg)
        ac = _mm(hi, sel16) + _mm(lo, sel16)
        se_all = jax.nn.sigmoid(ste[:, :_NFL * _BD])
        lde_l.append(jnp.sum(jnp.log(se_all)).reshape(1, 1))
        p = se_all
        for k in (_BD, 2 * _BD, 4 * _BD, 8 * _BD):
            p = p * jnp.concatenate(
                [p[:, k:], jnp.ones((_E, k), _F32)], axis=1)
        q = jnp.concatenate(
            [p[:, _BD:], jnp.ones((_E, _BD), _F32)], axis=1)
        tq = ste[:, _NFL * _BD:] * q
        for k in (_BD, 2 * _BD, 4 * _BD, 8 * _BD):
            tq = tq + jnp.concatenate(
                [tq[:, k:], jnp.zeros((_E, k), _F32)], axis=1)
        ze_ref[b] = ac * p[:, :_BD] + tq[:, :_BD]
    lde_ref[...] = jnp.concatenate(lde_l, axis=0)


def kernel(inp_node_features, inp_adj_features, rgcn_w0, rgcn_w,
           node_st_w, node_st_b, edge_st_w, edge_st_b):
    x = inp_node_features
    adj = inp_adj_features
    b = x.shape[0]
    xp = jnp.pad(x, ((0, 0), (0, _NP - _MAX), (0, 0)))
    adjs = jnp.pad(adj, ((0, 0), (0, 0), (0, _NP - _MAX), (0, _NP - _MAX)))
    adjs = adjs.reshape(b, _KC, _NP)

    w0st = rgcn_w0.reshape(_BD * _ND, _NH)
    wst = rgcn_w.reshape(2, _BD * _NH, _NH)

    wn = jnp.concatenate(
        [node_st_w[:, :, :_ND].transpose(1, 0, 2).reshape(_NH, _NFL * _ND),
         node_st_w[:, :, _ND:].transpose(1, 0, 2).reshape(_NH, _NFL * _ND)],
        axis=1)
    bn = jnp.concatenate(
        [node_st_b[:, :_ND].reshape(-1) + 2.0,
         node_st_b[:, _ND:].reshape(-1)],
        axis=0).reshape(1, 2 * _NFL * _ND)

    def epack(wslice):
        return jnp.concatenate(
            [wslice[:, :, :_BD].transpose(1, 0, 2).reshape(_NH, _NFL * _BD),
             wslice[:, :, _BD:].transpose(1, 0, 2).reshape(_NH, _NFL * _BD)],
            axis=1)

    wg = epack(edge_st_w[:, :_NH, :])
    ws = epack(edge_st_w[:, _NH:2 * _NH, :])
    wd = epack(edge_st_w[:, 2 * _NH:, :])
    be = jnp.concatenate(
        [edge_st_b[:, :_BD].reshape(-1) + 2.0,
         edge_st_b[:, _BD:].reshape(-1)],
        axis=0).reshape(1, 2 * _NFL * _BD)

    bb = 16
    grid = (b // bb,)

    def bs_b(shape):
        n = len(shape)
        return pl.BlockSpec((bb,) + shape, lambda i, n=n: (i,) + (0,) * n)

    def bs_w(shape):
        n = len(shape)
        return pl.BlockSpec(shape, lambda i, n=n: (0,) * n)

    out_shape = [
        jax.ShapeDtypeStruct((b, _MAX, _ND), jnp.float32),
        jax.ShapeDtypeStruct((b, _E, _BD), jnp.float32),
        jax.ShapeDtypeStruct((b, 1), jnp.float32),
        jax.ShapeDtypeStruct((b, 1), jnp.float32),
    ]
    out_specs = [
        bs_b((_MAX, _ND)),
        bs_b((_E, _BD)),
        pl.BlockSpec((bb, 1), lambda i: (i, 0)),
        pl.BlockSpec((bb, 1), lambda i: (i, 0)),
    ]
    in_specs = [
        bs_b((_NP, _ND)),
        bs_b((_KC, _NP)),
        bs_w(w0st.shape),
        bs_w(wst.shape),
        bs_w(wn.shape),
        bs_w(bn.shape),
        bs_w(wg.shape),
        bs_w(wd.shape),
        bs_w(ws.shape),
        bs_w(be.shape),
        bs_w(_GDST.shape),
        bs_w(_GSRC.shape),
        bs_w(_LINCL.shape),
        bs_w(_GSRCT.shape),
        bs_w(_SEL.shape),
        bs_w(_MASKN.shape),
    ]
    zn, ze, ldn, lde = pl.pallas_call(
        _body,
        grid=grid,
        in_specs=in_specs,
        out_specs=out_specs,
        out_shape=out_shape,
        compiler_params=pltpu.CompilerParams(
            dimension_semantics=("arbitrary",)),
    )(xp, adjs, w0st, wst, wn, bn, wg, wd, ws, be,
      jnp.asarray(_GDST), jnp.asarray(_GSRC), jnp.asarray(_LINCL),
      jnp.asarray(_GSRCT), jnp.asarray(_SEL), jnp.asarray(_MASKN))
    return (zn.reshape(b, -1), ze.reshape(b, -1),
            ldn.reshape(b), lde.reshape(b))


# speedup vs baseline: 1.4128x; 1.0010x over previous
import numpy as np
import jax
import jax.numpy as jnp
from jax import lax
from jax.experimental import pallas as pl
from jax.experimental.pallas import tpu as pltpu

_MAX = 38
_NP = 40
_EU = 12
_ND = 9
_BD = 4
_KC = _BD * _NP
_NH = 128
_NFL = 12


def _static_consts():
    src, dst = [], []
    for i in range(_MAX):
        start = 0 if i < _EU else i - _EU
        tot = i if i < _EU else _EU
        for j in range(tot):
            src.append(start + j)
            dst.append(i)
    src = np.asarray(src)
    dst = np.asarray(dst)
    e = src.shape[0]
    gdst = np.zeros((e, _NP), np.float32)
    gdst[np.arange(e), dst] = 1.0
    gsrc = np.zeros((e, _NP), np.float32)
    gsrc[np.arange(e), src] = 1.0
    lincl = np.tril(np.ones((_NP, _NP), np.float32))
    lincl[_MAX:, :] = 0.0
    gsrct = np.tile(gsrc, (1, _BD))
    sel = np.zeros((_KC, _BD), np.float32)
    for r in range(_BD):
        sel[r * _NP:(r + 1) * _NP, r] = 1.0
    mask_n = np.zeros((1, _NP, 1), np.float32)
    mask_n[0, :_MAX, 0] = 1.0
    return gdst, gsrc, lincl, gsrct, sel, mask_n


_GDST, _GSRC, _LINCL, _GSRCT, _SEL, _MASKN = _static_consts()
_E = _GDST.shape[0]
_BF = jnp.bfloat16
_F32 = jnp.float32


def _mm(a, b):
    return jnp.dot(a, b, preferred_element_type=_F32)


def _mm_t(a, b):
    return lax.dot_general(a, b, (((1,), (1,)), ((), ())),
                           preferred_element_type=_F32)


def _split(u):
    hi = u.astype(_BF)
    lo = (u - hi.astype(_F32)).astype(_BF)
    return hi, lo


def _xdot(g16, u):
    hi, lo = _split(u)
    return _mm(g16, hi) + _mm(g16, lo)


def _xdot_t(g16, u):
    hi, lo = _split(u)
    return _mm_t(g16, hi) + _mm_t(g16, lo)


def _body(x_ref, adjs_ref, w0st_ref, wst_ref, wn_ref, bn_ref,
          wg_ref, wd_ref, ws_ref, be_ref,
          gdst_ref, gsrc_ref, lincl_ref, gsrct_ref, sel_ref, maskn_ref,
          zn_ref, ze_ref, ldn_ref, lde_ref):
    bb = x_ref.shape[0]

    a16 = [adjs_ref[b].astype(_BF) for b in range(bb)]

    def rgcn_layer(hb16, wst16):
        out = []
        for b in range(bb):
            m = _mm(a16[b], hb16[b])
            mcat = jnp.concatenate(
                [m[r * _NP:(r + 1) * _NP, :] for r in range(_BD)], axis=1)
            out.append(jnp.maximum(_mm(mcat.astype(_BF), wst16), 0.0))
        return out

    hb = rgcn_layer([x_ref[b].astype(_BF) for b in range(bb)],
                    w0st_ref[...].astype(_BF))
    for l in range(2):
        hb = rgcn_layer([h.astype(_BF) for h in hb],
                        wst_ref[l].astype(_BF))

    h16 = [h.astype(_BF) for h in hb]
    l16 = lincl_ref[...].astype(_BF)
    cs = jnp.concatenate([_mm(l16, h16[b]) for b in range(bb)], axis=0)
    hs16 = jnp.concatenate(h16, axis=0)
    gs = cs - hs16.astype(_F32)

    stn = _mm(gs.astype(_BF), wn_ref[...].astype(_BF)) + bn_ref[0]
    s_all = jax.nn.sigmoid(stn[:, :_NFL * _ND])
    logs = jnp.log(s_all).reshape(bb, _NP, _NFL * _ND) * maskn_ref[...]
    ldn_ref[...] = jnp.sum(logs, axis=(1, 2)).reshape(bb, 1)
    z = x_ref[...].reshape(bb * _NP, _ND)
    for l in range(_NFL):
        z = (z * s_all[:, l * _ND:(l + 1) * _ND]
             + stn[:, _NFL * _ND + l * _ND:_NFL * _ND + (l + 1) * _ND])
    zn_ref[...] = z.reshape(bb, _NP, _ND)[:, :_MAX, :]

    u = (_mm(cs.astype(_BF), wg_ref[...].astype(_BF))
         + _mm(hs16, wd_ref[...].astype(_BF)))
    v = _mm(hs16, ws_ref[...].astype(_BF))
    gdst16 = gdst_ref[...].astype(_BF)
    gsrc16 = gsrc_ref[...].astype(_BF)
    gsrct = gsrct_ref[...]
    sel16 = sel_ref[...].astype(_BF)
    be = be_ref[0]
    lde_l = []
    for b in range(bb):
        ub = u[b * _NP:(b + 1) * _NP, :]
        vb = v[b * _NP:(b + 1) * _NP, :]
        ste = _xdot(gdst16, ub) + _xdot(gsrc16, vb) + be
        rmat = _xdot_t(gdst16, adjs_ref[b])
        rg = rmat * gsrct
        hi, lo = _split(rg)
        ac = _mm(hi, sel16) + _mm(lo, sel16)
        se_all = jax.nn.sigmoid(ste[:, :_NFL * _BD])
        lde_l.append(jnp.sum(jnp.log(se_all)).reshape(1, 1))
        p = se_all
        for k in (_BD, 2 * _BD, 4 * _BD, 8 * _BD):
            p = p * jnp.concatenate(
                [p[:, k:], jnp.ones((_E, k), _F32)], axis=1)
        q = jnp.concatenate(
            [p[:, _BD:], jnp.ones((_E, _BD), _F32)], axis=1)
        tq = ste[:, _NFL * _BD:] * q
        for k in (_BD, 2 * _BD, 4 * _BD, 8 * _BD):
            tq = tq + jnp.concatenate(
                [tq[:, k:], jnp.zeros((_E, k), _F32)], axis=1)
        ze_ref[b] = ac * p[:, :_BD] + tq[:, :_BD]
    lde_ref[...] = jnp.concatenate(lde_l, axis=0)


def kernel(inp_node_features, inp_adj_features, rgcn_w0, rgcn_w,
           node_st_w, node_st_b, edge_st_w, edge_st_b):
    x = inp_node_features
    adj = inp_adj_features
    b = x.shape[0]
    xp = jnp.pad(x, ((0, 0), (0, _NP - _MAX), (0, 0)))
    adjs = jnp.pad(adj, ((0, 0), (0, 0), (0, _NP - _MAX), (0, _NP - _MAX)))
    adjs = adjs.reshape(b, _KC, _NP)

    w0st = rgcn_w0.reshape(_BD * _ND, _NH)
    wst = rgcn_w.reshape(2, _BD * _NH, _NH)

    wn = jnp.concatenate(
        [node_st_w[:, :, :_ND].transpose(1, 0, 2).reshape(_NH, _NFL * _ND),
         node_st_w[:, :, _ND:].transpose(1, 0, 2).reshape(_NH, _NFL * _ND)],
        axis=1)
    bn = jnp.concatenate(
        [node_st_b[:, :_ND].reshape(-1) + 2.0,
         node_st_b[:, _ND:].reshape(-1)],
        axis=0).reshape(1, 2 * _NFL * _ND)

    def epack(wslice):
        return jnp.concatenate(
            [wslice[:, :, :_BD].transpose(1, 0, 2).reshape(_NH, _NFL * _BD),
             wslice[:, :, _BD:].transpose(1, 0, 2).reshape(_NH, _NFL * _BD)],
            axis=1)

    wg = epack(edge_st_w[:, :_NH, :])
    ws = epack(edge_st_w[:, _NH:2 * _NH, :])
    wd = epack(edge_st_w[:, 2 * _NH:, :])
    be = jnp.concatenate(
        [edge_st_b[:, :_BD].reshape(-1) + 2.0,
         edge_st_b[:, _BD:].reshape(-1)],
        axis=0).reshape(1, 2 * _NFL * _BD)

    bb = 16
    grid = (b // bb,)

    def bs_b(shape):
        n = len(shape)
        return pl.BlockSpec((bb,) + shape, lambda i, n=n: (i,) + (0,) * n)

    def bs_w(shape):
        n = len(shape)
        return pl.BlockSpec(shape, lambda i, n=n: (0,) * n)

    out_shape = [
        jax.ShapeDtypeStruct((b, _MAX, _ND), jnp.float32),
        jax.ShapeDtypeStruct((b, _E, _BD), jnp.float32),
        jax.ShapeDtypeStruct((b, 1), jnp.float32),
        jax.ShapeDtypeStruct((b, 1), jnp.float32),
    ]
    out_specs = [
        bs_b((_MAX, _ND)),
        bs_b((_E, _BD)),
        pl.BlockSpec((bb, 1), lambda i: (i, 0)),
        pl.BlockSpec((bb, 1), lambda i: (i, 0)),
    ]
    in_specs = [
        bs_b((_NP, _ND)),
        bs_b((_KC, _NP)),
        bs_w(w0st.shape),
        bs_w(wst.shape),
        bs_w(wn.shape),
        bs_w(bn.shape),
        bs_w(wg.shape),
        bs_w(wd.shape),
        bs_w(ws.shape),
        bs_w(be.shape),
        bs_w(_GDST.shape),
        bs_w(_GSRC.shape),
        bs_w(_LINCL.shape),
        bs_w(_GSRCT.shape),
        bs_w(_SEL.shape),
        bs_w(_MASKN.shape),
    ]
    zn, ze, ldn, lde = pl.pallas_call(
        _body,
        grid=grid,
        in_specs=in_specs,
        out_specs=out_specs,
        out_shape=out_shape,
        compiler_params=pltpu.CompilerParams(
            dimension_semantics=("arbitrary",)),
    )(xp, adjs, w0st, wst, wn, bn, wg, wd, ws, be,
      jnp.asarray(_GDST), jnp.asarray(_GSRC), jnp.asarray(_LINCL),
      jnp.asarray(_GSRCT), jnp.asarray(_SEL), jnp.asarray(_MASKN))
    return (zn.reshape(b, -1), ze.reshape(b, -1),
            ldn.reshape(b), lde.reshape(b))
